# baseline (speedup 1.0000x reference)
import functools
import jax
import jax.numpy as jnp
from jax import lax
from jax.experimental import pallas as pl
from jax.experimental.pallas import tpu as pltpu
from jax.experimental.pallas import tpu_sc as plsc

B = 4096
L = 200
E = 64
LP1 = L + 1
VOCAB = 1000000

NC = 2
NS = 16
NW = NC * NS
BPW = B // NW
SRB = 8
SR = BPW // SRB
G = 2
RPS = SRB // G
R = BPW // G
S1 = 104
S2 = L - S1


def _body(x_hbm, subject_hbm, table_hbm, subject_table_hbm,
          emb_hbm, subj_hbm,
          idx0, idx1, gbuf, ssem, gsem0, gsem1, wsem0, wsem1):
    wid = lax.axis_index("s") * NC + lax.axis_index("c")
    base = wid * BPW

    idxs = (idx0, idx1)
    gsems = (gsem0, gsem1)
    wsems = (wsem0, wsem1)

    sidx = idx1.at[pl.ds(0, BPW)]
    srow = gbuf.at[0, 0, pl.ds(0, BPW)]
    pltpu.sync_copy(subject_hbm.at[pl.ds(base, BPW)], sidx)
    pltpu.async_copy(subject_table_hbm.at[sidx], srow, ssem).wait()
    pltpu.async_copy(srow, subj_hbm.at[pl.ds(base, BPW)], ssem).wait()

    def load_idx(sr_dyn, p):
        pltpu.sync_copy(x_hbm.at[pl.ds((base + sr_dyn * SRB) * L, SRB * L)],
                        idxs[p])

    def start_round(rr_dyn, lb0, ip, q):
        for g in range(G):
            for off, n in ((0, S1), (S1, S2)):
                pltpu.async_copy(
                    table_hbm.at[idxs[ip].at[pl.ds((lb0 + g) * L + off, n)]],
                    gbuf.at[q, g, pl.ds(off, n)],
                    gsems[q])

    def wait_gathers(q):
        for g in range(G):
            for off, n in ((0, S1), (S1, S2)):
                pltpu.make_async_copy(
                    table_hbm.at[idxs[0].at[pl.ds(off, n)]],
                    gbuf.at[q, g, pl.ds(off, n)],
                    gsems[q]).wait()

    def start_writes(rr_dyn, q):
        for g in range(G):
            pltpu.async_copy(gbuf.at[q, g],
                             emb_hbm.at[pl.ds((base + rr_dyn * G + g) * L, L)],
                             wsems[q])

    def wait_writes(q):
        for g in range(G):
            pltpu.make_async_copy(gbuf.at[q, g], emb_hbm.at[pl.ds(0, L)],
                                  wsems[q]).wait()

    load_idx(0, 0)
    start_round(0, 0, 0, 0)

    @pl.loop(0, SR, step=2)
    def _srs(ssr):
        for sp in (0, 1):
            sr = ssr + sp
            for rl in range(RPS):
                rr = sr * RPS + rl
                q = rl % 2
                nxt = 1 - q

                @pl.when(rr + 1 < R)
                def _():
                    @pl.when(rr >= 1)
                    def _():
                        wait_writes(nxt)
                    if rl == RPS - 1:
                        load_idx(sr + 1, 1 - sp)
                        start_round(rr + 1, 0, 1 - sp, nxt)
                    else:
                        start_round(rr + 1, (rl + 1) * G, sp, nxt)

                wait_gathers(q)
                start_writes(rr, q)

    wait_writes(0)
    wait_writes(1)


@jax.jit
def _run(x, subject, table, subject_table):
    tableP = jnp.concatenate([table, table], axis=1)
    stP = jnp.concatenate([subject_table, subject_table], axis=1)
    xflat = x.reshape(B * L)
    kern = functools.partial(
        pl.kernel,
        out_type=(
            jax.ShapeDtypeStruct((B * L, 128), jnp.float32),
            jax.ShapeDtypeStruct((B, 128), jnp.float32),
        ),
        mesh=plsc.VectorSubcoreMesh(
            core_axis_name="c", subcore_axis_name="s",
            num_cores=NC, num_subcores=NS),
        scratch_types=[
            pltpu.VMEM((SRB * L,), jnp.int32),
            pltpu.VMEM((SRB * L,), jnp.int32),
            pltpu.VMEM((2, G, L, 128), jnp.float32),
            pltpu.SemaphoreType.DMA,
            pltpu.SemaphoreType.DMA,
            pltpu.SemaphoreType.DMA,
            pltpu.SemaphoreType.DMA,
            pltpu.SemaphoreType.DMA,
        ],
        compiler_params=pltpu.CompilerParams(use_tc_tiling_on_sc=True),
    )(_body)
    emb, subj = kern(xflat, subject, tableP, stP)
    out0 = subj[:, None, :E]
    out1 = emb[:, :E].reshape(B, L, E)
    return jnp.concatenate([out0, out1], axis=1)


def kernel(x, subject, table, subject_table):
    return _run(x, subject, table, subject_table)


# speedup vs baseline: 1.0352x; 1.0352x over previous
import functools
import jax
import jax.numpy as jnp
from jax import lax
from jax.experimental import pallas as pl
from jax.experimental.pallas import tpu as pltpu
from jax.experimental.pallas import tpu_sc as plsc

B = 4096
L = 200
E = 64
LP1 = L + 1

NC = 2
NS = 16
NW = NC * NS
BPW = B // NW
G = 2
R = BPW // G
S1 = 104
S2 = L - S1


def _body(x_hbm, subject_hbm, table_hbm, subject_table_hbm, out_hbm,
          idx_v, gbuf, sidx_v, srow_v, ssem, gsem0, gsem1, wsem0, wsem1):
    wid = lax.axis_index("s") * NC + lax.axis_index("c")
    base = wid * BPW

    pltpu.sync_copy(subject_hbm.at[pl.ds(base, BPW)], sidx_v)
    pltpu.async_copy(subject_table_hbm.at[sidx_v], srow_v, ssem).wait()

    gsems = (gsem0, gsem1)
    wsems = (wsem0, wsem1)

    def start_round(r, q):
        pltpu.sync_copy(x_hbm.at[pl.ds(base + r * G, G)], idx_v.at[q])
        for g in range(G):
            for off, n in ((0, S1), (S1, S2)):
                pltpu.async_copy(
                    table_hbm.at[idx_v.at[q, g, pl.ds(off, n)]],
                    gbuf.at[q, g, pl.ds(1 + off, n)],
                    gsems[q])

    def wait_gathers(q):
        for g in range(G):
            for off, n in ((0, S1), (S1, S2)):
                pltpu.make_async_copy(
                    table_hbm.at[idx_v.at[q, g, pl.ds(off, n)]],
                    gbuf.at[q, g, pl.ds(1 + off, n)],
                    gsems[q]).wait()

    def start_writes(r, q):
        for g in range(G):
            for k in range(E // 16):
                gbuf[q, g, 0, pl.ds(k * 16, 16)] = (
                    srow_v[r * G + g, pl.ds(k * 16, 16)])
            pltpu.async_copy(gbuf.at[q, g], out_hbm.at[base + r * G + g],
                             wsems[q])

    def wait_writes(q):
        for g in range(G):
            pltpu.make_async_copy(gbuf.at[q, g], out_hbm.at[0],
                                  wsems[q]).wait()

    start_round(0, 0)

    @pl.loop(0, R, step=2)
    def _rounds(r):
        for q in (0, 1):
            rr = r + q
            nxt = 1 - q

            @pl.when(rr + 1 < R)
            def _():
                @pl.when(rr >= 1)
                def _():
                    wait_writes(nxt)
                start_round(rr + 1, nxt)

            wait_gathers(q)
            start_writes(rr, q)

    wait_writes(0)
    wait_writes(1)


@jax.jit
def _run(x, subject, table, subject_table):
    kern = functools.partial(
        pl.kernel,
        out_type=jax.ShapeDtypeStruct((B, LP1, E), jnp.float32),
        mesh=plsc.VectorSubcoreMesh(
            core_axis_name="c", subcore_axis_name="s",
            num_cores=NC, num_subcores=NS),
        scratch_types=[
            pltpu.VMEM((2, G, L), jnp.int32),
            pltpu.VMEM((2, G, LP1, E), jnp.float32),
            pltpu.VMEM((BPW,), jnp.int32),
            pltpu.VMEM((BPW, E), jnp.float32),
            pltpu.SemaphoreType.DMA,
            pltpu.SemaphoreType.DMA,
            pltpu.SemaphoreType.DMA,
            pltpu.SemaphoreType.DMA,
            pltpu.SemaphoreType.DMA,
        ],
        compiler_params=pltpu.CompilerParams(use_tc_tiling_on_sc=False),
    )(_body)
    return kern(x, subject, table, subject_table)


def kernel(x, subject, table, subject_table):
    return _run(x, subject, table, subject_table)
